# Initial kernel scaffold; baseline (speedup 1.0000x reference)
import jax
import jax.numpy as jnp
from jax.experimental import pallas as pl


def kernel(x, table):
    raise NotImplementedError("write your pallas kernel here")


# speedup vs baseline: 1.2960x; 1.2960x over previous
import functools

import jax
import jax.numpy as jnp
from jax import lax
from jax.experimental import pallas as pl
from jax.experimental.pallas import tpu as pltpu
from jax.experimental.pallas import tpu_sc as plsc


def _make_gather(n_workers, n_chunks, chunk, d, d_pad):
    mesh = plsc.VectorSubcoreMesh(core_axis_name="c", subcore_axis_name="s")
    per_w = n_chunks * chunk

    @functools.partial(
        pl.kernel,
        out_type=jax.ShapeDtypeStruct((n_workers * per_w, d_pad), jnp.float32),
        mesh=mesh,
        scratch_types=[
            pltpu.VMEM((n_chunks, chunk), jnp.int32),
            pltpu.VMEM((chunk, d_pad), jnp.float32),
            pltpu.SemaphoreType.DMA,
        ],
    )
    def glove_gather(idx_hbm, table_hbm, out_hbm, idx_v, buf, sem):
        n_cores = mesh.num_cores
        wid = lax.axis_index("s") * n_cores + lax.axis_index("c")
        row_base = wid * per_w
        pltpu.sync_copy(idx_hbm.at[wid], idx_v)

        def body(g, carry):
            pltpu.async_copy(table_hbm.at[idx_v.at[g]], buf, sem).wait()
            pltpu.sync_copy(buf, out_hbm.at[pl.ds(row_base + g * chunk, chunk)])
            return carry

        lax.fori_loop(0, n_chunks, body, 0)

    return glove_gather


def kernel(x, table):
    b, s = x.shape
    v, d = table.shape
    d_pad = 384
    n = b * s
    n_workers = 32
    chunk = 128
    per_w = n // n_workers
    n_chunks = per_w // chunk
    idx = x.reshape(n_workers, n_chunks, chunk).astype(jnp.int32)
    table_pad = jnp.pad(table, ((0, 0), (0, d_pad - d)))
    out = _make_gather(n_workers, n_chunks, chunk, d, d_pad)(idx, table_pad)
    return out[:, :d].reshape(b, s, d)
